# baseline (speedup 1.0000x reference)
import functools
import math

import jax
import jax.numpy as jnp
from jax import lax
from jax.experimental import pallas as pl
from jax.experimental.pallas import tpu as pltpu, tpu_sc as plsc

N_HEADS = 16
D_K = 64
D_V = 64
TOP_K = 32
NEG_INF = float("-inf")


def _proj_body(q_ref, k_ref, v_ref, wq_ref, bq_ref, wk_ref, bk_ref, wv_ref, bv_ref,
               qs_ref, ks_ref, vs_ref):
    qs_ref[...] = jnp.dot(q_ref[...], wq_ref[...], preferred_element_type=jnp.float32) + bq_ref[...]
    ks_ref[...] = jnp.dot(k_ref[...], wk_ref[...], preferred_element_type=jnp.float32) + bk_ref[...]
    vs_ref[...] = jnp.dot(v_ref[...], wv_ref[...], preferred_element_type=jnp.float32) + bv_ref[...]


def _scores_body(qs_ref, ks_ref, o_ref, *, bq, hpp):
    q = qs_ref[...]
    k = ks_ref[...]
    scale = 1.0 / math.sqrt(D_K)
    outs = []
    for j in range(hpp):
        s = lax.dot_general(q[:, j * D_K:(j + 1) * D_K], k[:, j * D_K:(j + 1) * D_K],
                            (((1,), (1,)), ((), ())),
                            preferred_element_type=jnp.float32) * scale
        outs.append(s[None, None])
    o_ref[...] = jnp.concatenate(outs, axis=0)


def _sc_threshold(scores, seq_len, n_rows):
    info = plsc.get_sparse_core_info()
    NC, NS, L = info.num_cores, info.num_subcores, info.num_lanes
    NW = NC * NS
    rows_per_w = n_rows // NW
    batch = 16
    n_batches = rows_per_w // batch
    n_vregs = seq_len // L

    mesh = plsc.VectorSubcoreMesh(core_axis_name="c", subcore_axis_name="s")

    nseg = 4
    vps = n_vregs // nseg
    segcap = vps * L + L
    survsz = nseg * segcap
    KSTAT = 4

    @functools.partial(
        pl.kernel,
        out_type=jax.ShapeDtypeStruct((n_rows,), jnp.float32),
        mesh=mesh,
        scratch_types=[
            pltpu.VMEM((batch, seq_len), jnp.float32),
            pltpu.VMEM((2 * (seq_len + 4 * L),), jnp.float32),
            pltpu.VMEM((rows_per_w,), jnp.float32),
            pltpu.SemaphoreType.DMA,
        ],
        compiler_params=pltpu.CompilerParams(needs_layout_passes=False),
    )
    def body(scores_hbm, out_hbm, rowbuf, survbuf, threshbuf, sem):
        wid = lax.axis_index("s") * NC + lax.axis_index("c")
        row0 = wid * rows_per_w
        lane0 = lax.iota(jnp.int32, L) == 0
        ninf = jnp.full((L,), NEG_INF, jnp.float32)

        def filt(r, arena):
            m_even = rowbuf[r, pl.ds(0, L)]
            m_odd = rowbuf[r, pl.ds(L, L)]
            for i in range(2, n_vregs, 2):
                m_even = jnp.maximum(m_even, rowbuf[r, pl.ds(i * L, L)])
                m_odd = jnp.maximum(m_odd, rowbuf[r, pl.ds((i + 1) * L, L)])
            lb = -jnp.max(-jnp.minimum(m_even, m_odd))
            lb_v = jnp.full((L,), lb, jnp.float32)

            for g in range(nseg):
                for j in range(1, KSTAT):
                    survbuf[pl.ds(arena + g * segcap + j * L, L)] = ninf

            cnts = [0] * nseg
            for i in range(vps):
                for g in range(nseg):
                    v = rowbuf[r, pl.ds((g * vps + i) * L, L)]
                    mask = v >= lb_v
                    plsc.store_compressed(
                        survbuf.at[pl.ds(arena + g * segcap + cnts[g], L)], v, mask=mask)
                    cnts[g] = cnts[g] + plsc.all_reduce_population_count(mask)[0]
            for g in range(nseg):
                survbuf[pl.ds(arena + g * segcap + cnts[g], L)] = ninf
            return cnts

        def select(arena, cnts, prow):
            def mk_merge(base):
                def merge(i, carry):
                    thi, tlo = carry
                    bs = jnp.sort(survbuf[pl.ds(base + i * L, L)])
                    x = jnp.sort(jnp.maximum(tlo, lax.rev(bs, (0,))))
                    rx = lax.rev(x, (0,))
                    return jnp.sort(jnp.maximum(thi, rx)), jnp.sort(jnp.minimum(thi, rx))
                return merge

            carry = (ninf, ninf)
            for g in range(nseg):
                m = mk_merge(arena + g * segcap)
                for j in range(KSTAT):
                    carry = m(j, carry)
            for g in range(nseg):
                nv = (cnts[g] + L - 1) // L
                carry = lax.fori_loop(KSTAT, nv, mk_merge(arena + g * segcap), carry)
            t = -jnp.max(-carry[1])
            prow_v = jnp.full((L,), prow, jnp.int32)
            plsc.store_scatter(
                threshbuf,
                [jnp.maximum(prow_v, 0)],
                jnp.full((L,), t, jnp.float32),
                mask=lane0 & (prow_v >= 0),
            )

        def do_batch(b, _):
            pltpu.async_copy(
                scores_hbm.at[pl.ds(row0 + b * batch, batch), :], rowbuf, sem
            ).wait()

            def do_row(r, carry):
                pc, prow = carry
                arena = (r % 2) * survsz
                cnts = filt(r, arena)
                select((1 - r % 2) * survsz, pc, prow)
                return tuple(cnts), b * batch + r

            zero = jnp.zeros((), jnp.int32)
            pc, prow = lax.fori_loop(
                0, batch, do_row, ((zero,) * nseg, -jnp.ones((), jnp.int32)))
            select(survsz, pc, prow)
            return 0

        lax.fori_loop(0, n_batches, do_batch, 0)
        pltpu.sync_copy(threshbuf, out_hbm.at[pl.ds(row0, rows_per_w)])

    return body(scores)


def _finish_body(qs_ref, ks_ref, vs_ref, t0_ref, t1_ref, o_ref, *, seq_len, bq, hpp):
    q = qs_ref[...]
    k = ks_ref[...]
    v = vs_ref[...]
    scale = 1.0 / math.sqrt(D_K)
    s = jnp.concatenate([
        lax.dot_general(k[:, j * D_K:(j + 1) * D_K], q[:, j * D_K:(j + 1) * D_K],
                        (((1,), (1,)), ((), ())),
                        preferred_element_type=jnp.float32)
        for j in range(hpp)
    ], axis=1) * scale

    t = jnp.concatenate([t0_ref[...].reshape(1, bq), t1_ref[...].reshape(1, bq)],
                        axis=1)
    w = jnp.where(s >= t, jnp.exp(s) - 1.0, 0.0)
    denom = jnp.sum(w, axis=0) + (float(seq_len) + 1e-8)
    outs = []
    for j in range(hpp):
        vj = v[:, j * D_V:(j + 1) * D_V]
        wj = w[:, j * bq:(j + 1) * bq]
        colsum = jnp.sum(vj, axis=0)
        num = lax.dot_general(wj, vj, (((0,), (0,)), ((), ())),
                              preferred_element_type=jnp.float32)
        outs.append((num + colsum[None, :]) / denom[j * bq:(j + 1) * bq, None])
    o_ref[...] = jnp.concatenate(outs, axis=1)


def kernel(Q, K, V, W_Q, b_Q, W_K, b_K, W_V, b_V):
    batch, seq_len, d_model = Q.shape
    d_out = W_Q.shape[1]
    n_heads = d_out // D_K
    q2 = Q.reshape(seq_len, d_model)
    k2 = K.reshape(seq_len, d_model)
    v2 = V.reshape(seq_len, d_model)

    sb = 256
    proj = pl.pallas_call(
        _proj_body,
        grid=(seq_len // sb,),
        in_specs=[
            pl.BlockSpec((sb, d_model), lambda i: (i, 0)),
            pl.BlockSpec((sb, d_model), lambda i: (i, 0)),
            pl.BlockSpec((sb, d_model), lambda i: (i, 0)),
            pl.BlockSpec((d_model, d_out), lambda i: (0, 0)),
            pl.BlockSpec((1, d_out), lambda i: (0, 0)),
            pl.BlockSpec((d_model, d_out), lambda i: (0, 0)),
            pl.BlockSpec((1, d_out), lambda i: (0, 0)),
            pl.BlockSpec((d_model, d_out), lambda i: (0, 0)),
            pl.BlockSpec((1, d_out), lambda i: (0, 0)),
        ],
        out_specs=[
            pl.BlockSpec((sb, d_out), lambda i: (i, 0)),
            pl.BlockSpec((sb, d_out), lambda i: (i, 0)),
            pl.BlockSpec((sb, d_out), lambda i: (i, 0)),
        ],
        out_shape=[jax.ShapeDtypeStruct((seq_len, d_out), jnp.float32)] * 3,
    )
    qs, ks, vs = proj(q2, k2, v2,
                      W_Q, b_Q.reshape(1, d_out),
                      W_K, b_K.reshape(1, d_out),
                      W_V, b_V.reshape(1, d_out))

    bq = 128
    hpp = 2
    nqb = seq_len // bq
    scores = pl.pallas_call(
        functools.partial(_scores_body, bq=bq, hpp=hpp),
        grid=(n_heads // hpp, nqb),
        in_specs=[
            pl.BlockSpec((bq, hpp * D_K), lambda h, i: (i, h)),
            pl.BlockSpec((seq_len, hpp * D_K), lambda h, i: (0, h)),
        ],
        out_specs=pl.BlockSpec((hpp, 1, bq, seq_len), lambda h, i: (h, i, 0, 0)),
        out_shape=jax.ShapeDtypeStruct((n_heads, nqb, bq, seq_len), jnp.float32),
    )(qs, ks)

    n_rows = n_heads * seq_len
    thresh = _sc_threshold(scores.reshape(n_rows, seq_len), seq_len, n_rows)
    thresh3 = thresh.reshape(n_heads * nqb, 1, bq)

    finish = pl.pallas_call(
        functools.partial(_finish_body, seq_len=seq_len, bq=bq, hpp=hpp),
        grid=(n_heads // hpp, nqb),
        in_specs=[
            pl.BlockSpec((bq, hpp * D_K), lambda h, i: (i, h)),
            pl.BlockSpec((seq_len, hpp * D_K), lambda h, i: (0, h)),
            pl.BlockSpec((seq_len, hpp * D_V), lambda h, i: (0, h)),
            pl.BlockSpec((1, 1, bq), lambda h, i, n=nqb: (2 * h * n + i, 0, 0)),
            pl.BlockSpec((1, 1, bq), lambda h, i, n=nqb: ((2 * h + 1) * n + i, 0, 0)),
        ],
        out_specs=pl.BlockSpec((bq, hpp * D_V), lambda h, i: (i, h)),
        out_shape=jax.ShapeDtypeStruct((seq_len, d_out), jnp.float32),
    )(qs, ks, vs, thresh3, thresh3)
    return finish.reshape(batch, seq_len, d_out)


# speedup vs baseline: 1.0404x; 1.0404x over previous
import functools
import math

import jax
import jax.numpy as jnp
from jax import lax
from jax.experimental import pallas as pl
from jax.experimental.pallas import tpu as pltpu, tpu_sc as plsc

N_HEADS = 16
D_K = 64
D_V = 64
TOP_K = 32
NEG_INF = float("-inf")


def _proj_body(q_ref, k_ref, v_ref, wq_ref, bq_ref, wk_ref, bk_ref, wv_ref, bv_ref,
               qs_ref, ks_ref, vs_ref):
    qs_ref[...] = jnp.dot(q_ref[...], wq_ref[...], preferred_element_type=jnp.float32) + bq_ref[...]
    ks_ref[...] = jnp.dot(k_ref[...], wk_ref[...], preferred_element_type=jnp.float32) + bk_ref[...]
    vs_ref[...] = jnp.dot(v_ref[...], wv_ref[...], preferred_element_type=jnp.float32) + bv_ref[...]


def _scores_body(qs_ref, ks_ref, o_ref, *, bq, hpp):
    q = qs_ref[...]
    k = ks_ref[...]
    scale = 1.0 / math.sqrt(D_K)
    outs = []
    for j in range(hpp):
        s = lax.dot_general(q[:, j * D_K:(j + 1) * D_K], k[:, j * D_K:(j + 1) * D_K],
                            (((1,), (1,)), ((), ())),
                            preferred_element_type=jnp.float32) * scale
        outs.append(s[None, None])
    o_ref[...] = jnp.concatenate(outs, axis=0)


def _sc_threshold(scores, seq_len, n_rows):
    info = plsc.get_sparse_core_info()
    NC, NS, L = info.num_cores, info.num_subcores, info.num_lanes
    NW = NC * NS
    rows_per_w = n_rows // NW
    batch = 16
    n_batches = rows_per_w // batch
    n_vregs = seq_len // L

    mesh = plsc.VectorSubcoreMesh(core_axis_name="c", subcore_axis_name="s")

    nseg = 4
    vps = n_vregs // nseg
    segcap = vps * L + L
    survsz = nseg * segcap
    KSTAT = 4

    @functools.partial(
        pl.kernel,
        out_type=jax.ShapeDtypeStruct((n_rows,), jnp.float32),
        mesh=mesh,
        scratch_types=[
            pltpu.VMEM((batch, seq_len), jnp.float32),
            pltpu.VMEM((2 * (seq_len + 4 * L),), jnp.float32),
            pltpu.VMEM((rows_per_w,), jnp.float32),
            pltpu.SemaphoreType.DMA,
        ],
        compiler_params=pltpu.CompilerParams(needs_layout_passes=False),
    )
    def body(scores_hbm, out_hbm, rowbuf, survbuf, threshbuf, sem):
        wid = lax.axis_index("s") * NC + lax.axis_index("c")
        row0 = wid * rows_per_w
        lane0 = lax.iota(jnp.int32, L) == 0
        ninf = jnp.full((L,), NEG_INF, jnp.float32)

        def filt(r, arena):
            m_even = rowbuf[r, pl.ds(0, L)]
            m_odd = rowbuf[r, pl.ds(L, L)]
            for i in range(2, n_vregs, 2):
                m_even = jnp.maximum(m_even, rowbuf[r, pl.ds(i * L, L)])
                m_odd = jnp.maximum(m_odd, rowbuf[r, pl.ds((i + 1) * L, L)])
            lb = -jnp.max(-jnp.minimum(m_even, m_odd))
            lb_v = jnp.full((L,), lb, jnp.float32)

            for g in range(nseg):
                for j in range(1, KSTAT):
                    survbuf[pl.ds(arena + g * segcap + j * L, L)] = ninf

            cnts = [0] * nseg
            for i in range(vps):
                for g in range(nseg):
                    v = rowbuf[r, pl.ds((g * vps + i) * L, L)]
                    mask = v >= lb_v
                    plsc.store_compressed(
                        survbuf.at[pl.ds(arena + g * segcap + cnts[g], L)], v, mask=mask)
                    cnts[g] = cnts[g] + plsc.all_reduce_population_count(mask)[0]
            for g in range(nseg):
                survbuf[pl.ds(arena + g * segcap + cnts[g], L)] = ninf
            return cnts

        def select(arena, cnts, prow):
            def absorb(carry, bs):
                thi, tlo = carry
                x = jnp.sort(jnp.maximum(tlo, lax.rev(bs, (0,))))
                rx = lax.rev(x, (0,))
                return jnp.sort(jnp.maximum(thi, rx)), jnp.sort(jnp.minimum(thi, rx))

            def mk_merge(base):
                def merge(i, carry):
                    return absorb(carry, jnp.sort(survbuf[pl.ds(base + i * L, L)]))
                return merge

            carries = [(ninf, ninf)] * nseg
            for j in range(KSTAT):
                for g in range(nseg):
                    carries[g] = mk_merge(arena + g * segcap)(j, carries[g])
            for g in range(nseg):
                nv = (cnts[g] + L - 1) // L
                carries[g] = lax.fori_loop(KSTAT, nv, mk_merge(arena + g * segcap),
                                           carries[g])
            c01 = absorb(absorb(carries[0], carries[1][0]), carries[1][1])
            c23 = absorb(absorb(carries[2], carries[3][0]), carries[3][1])
            tlo = absorb(absorb(c01, c23[0]), c23[1])[1]
            t = -jnp.max(-tlo)
            prow_v = jnp.full((L,), prow, jnp.int32)
            plsc.store_scatter(
                threshbuf,
                [jnp.maximum(prow_v, 0)],
                jnp.full((L,), t, jnp.float32),
                mask=lane0 & (prow_v >= 0),
            )

        def do_batch(b, carry):
            pltpu.async_copy(
                scores_hbm.at[pl.ds(row0 + b * batch, batch), :], rowbuf, sem
            ).wait()

            def do_row(r, carry):
                pc, prow = carry
                select((1 - r % 2) * survsz, pc, prow)
                cnts = filt(r, (r % 2) * survsz)
                return tuple(cnts), b * batch + r

            return lax.fori_loop(0, batch, do_row, carry)

        zero = jnp.zeros((), jnp.int32)
        pc, prow = lax.fori_loop(
            0, n_batches, do_batch, ((zero,) * nseg, -jnp.ones((), jnp.int32)))
        select(survsz, pc, prow)
        pltpu.sync_copy(threshbuf, out_hbm.at[pl.ds(row0, rows_per_w)])

    return body(scores)


def _finish_body(qs_ref, ks_ref, vs_ref, t0_ref, t1_ref, o_ref, *, seq_len, bq, hpp):
    q = qs_ref[...]
    k = ks_ref[...]
    v = vs_ref[...]
    scale = 1.0 / math.sqrt(D_K)
    s = jnp.concatenate([
        lax.dot_general(k[:, j * D_K:(j + 1) * D_K], q[:, j * D_K:(j + 1) * D_K],
                        (((1,), (1,)), ((), ())),
                        preferred_element_type=jnp.float32)
        for j in range(hpp)
    ], axis=1) * scale

    t = jnp.concatenate([t0_ref[...].reshape(1, bq), t1_ref[...].reshape(1, bq)],
                        axis=1)
    w = jnp.where(s >= t, jnp.exp(s) - 1.0, 0.0)
    denom = jnp.sum(w, axis=0) + (float(seq_len) + 1e-8)
    outs = []
    for j in range(hpp):
        vj = v[:, j * D_V:(j + 1) * D_V]
        wj = w[:, j * bq:(j + 1) * bq]
        colsum = jnp.sum(vj, axis=0)
        num = lax.dot_general(wj, vj, (((0,), (0,)), ((), ())),
                              preferred_element_type=jnp.float32)
        outs.append((num + colsum[None, :]) / denom[j * bq:(j + 1) * bq, None])
    o_ref[...] = jnp.concatenate(outs, axis=1)


def kernel(Q, K, V, W_Q, b_Q, W_K, b_K, W_V, b_V):
    batch, seq_len, d_model = Q.shape
    d_out = W_Q.shape[1]
    n_heads = d_out // D_K
    q2 = Q.reshape(seq_len, d_model)
    k2 = K.reshape(seq_len, d_model)
    v2 = V.reshape(seq_len, d_model)

    sb = 256
    proj = pl.pallas_call(
        _proj_body,
        grid=(seq_len // sb,),
        in_specs=[
            pl.BlockSpec((sb, d_model), lambda i: (i, 0)),
            pl.BlockSpec((sb, d_model), lambda i: (i, 0)),
            pl.BlockSpec((sb, d_model), lambda i: (i, 0)),
            pl.BlockSpec((d_model, d_out), lambda i: (0, 0)),
            pl.BlockSpec((1, d_out), lambda i: (0, 0)),
            pl.BlockSpec((d_model, d_out), lambda i: (0, 0)),
            pl.BlockSpec((1, d_out), lambda i: (0, 0)),
            pl.BlockSpec((d_model, d_out), lambda i: (0, 0)),
            pl.BlockSpec((1, d_out), lambda i: (0, 0)),
        ],
        out_specs=[
            pl.BlockSpec((sb, d_out), lambda i: (i, 0)),
            pl.BlockSpec((sb, d_out), lambda i: (i, 0)),
            pl.BlockSpec((sb, d_out), lambda i: (i, 0)),
        ],
        out_shape=[jax.ShapeDtypeStruct((seq_len, d_out), jnp.float32)] * 3,
    )
    qs, ks, vs = proj(q2, k2, v2,
                      W_Q, b_Q.reshape(1, d_out),
                      W_K, b_K.reshape(1, d_out),
                      W_V, b_V.reshape(1, d_out))

    bq = 128
    hpp = 2
    nqb = seq_len // bq
    scores = pl.pallas_call(
        functools.partial(_scores_body, bq=bq, hpp=hpp),
        grid=(n_heads // hpp, nqb),
        in_specs=[
            pl.BlockSpec((bq, hpp * D_K), lambda h, i: (i, h)),
            pl.BlockSpec((seq_len, hpp * D_K), lambda h, i: (0, h)),
        ],
        out_specs=pl.BlockSpec((hpp, 1, bq, seq_len), lambda h, i: (h, i, 0, 0)),
        out_shape=jax.ShapeDtypeStruct((n_heads, nqb, bq, seq_len), jnp.float32),
    )(qs, ks)

    n_rows = n_heads * seq_len
    thresh = _sc_threshold(scores.reshape(n_rows, seq_len), seq_len, n_rows)
    thresh3 = thresh.reshape(n_heads * nqb, 1, bq)

    finish = pl.pallas_call(
        functools.partial(_finish_body, seq_len=seq_len, bq=bq, hpp=hpp),
        grid=(n_heads // hpp, nqb),
        in_specs=[
            pl.BlockSpec((bq, hpp * D_K), lambda h, i: (i, h)),
            pl.BlockSpec((seq_len, hpp * D_K), lambda h, i: (0, h)),
            pl.BlockSpec((seq_len, hpp * D_V), lambda h, i: (0, h)),
            pl.BlockSpec((1, 1, bq), lambda h, i, n=nqb: (2 * h * n + i, 0, 0)),
            pl.BlockSpec((1, 1, bq), lambda h, i, n=nqb: ((2 * h + 1) * n + i, 0, 0)),
        ],
        out_specs=pl.BlockSpec((bq, hpp * D_V), lambda h, i: (i, h)),
        out_shape=jax.ShapeDtypeStruct((seq_len, d_out), jnp.float32),
    )(qs, ks, vs, thresh3, thresh3)
    return finish.reshape(batch, seq_len, d_out)


# speedup vs baseline: 1.1646x; 1.1194x over previous
import functools
import math

import jax
import jax.numpy as jnp
from jax import lax
from jax.experimental import pallas as pl
from jax.experimental.pallas import tpu as pltpu, tpu_sc as plsc

N_HEADS = 16
D_K = 64
D_V = 64
TOP_K = 32
NEG_INF = float("-inf")


def _proj_body(q_ref, k_ref, v_ref, wq_ref, bq_ref, wk_ref, bk_ref, wv_ref, bv_ref,
               qs_ref, ks_ref, vs_ref):
    qs_ref[...] = jnp.dot(q_ref[...], wq_ref[...], preferred_element_type=jnp.float32) + bq_ref[...]
    ks_ref[...] = jnp.dot(k_ref[...], wk_ref[...], preferred_element_type=jnp.float32) + bk_ref[...]
    vs_ref[...] = jnp.dot(v_ref[...], wv_ref[...], preferred_element_type=jnp.float32) + bv_ref[...]


def _scores_body(qs_ref, ks_ref, o_ref, *, bq, hpp):
    q = qs_ref[...]
    k = ks_ref[...]
    scale = 1.0 / math.sqrt(D_K)
    outs = []
    for j in range(hpp):
        s = lax.dot_general(q[:, j * D_K:(j + 1) * D_K], k[:, j * D_K:(j + 1) * D_K],
                            (((1,), (1,)), ((), ())),
                            preferred_element_type=jnp.float32) * scale
        outs.append(s[None, None])
    o_ref[...] = jnp.concatenate(outs, axis=0)


def _sc_threshold(scores, seq_len, n_rows):
    info = plsc.get_sparse_core_info()
    NC, NS, L = info.num_cores, info.num_subcores, info.num_lanes
    NW = NC * NS
    rows_per_w = n_rows // NW
    batch = 16
    n_batches = rows_per_w // batch
    n_vregs = seq_len // L

    mesh = plsc.VectorSubcoreMesh(core_axis_name="c", subcore_axis_name="s")

    nseg = 4
    vps = n_vregs // nseg
    segcap = vps * L + L
    survsz = nseg * segcap

    @functools.partial(
        pl.kernel,
        out_type=jax.ShapeDtypeStruct((n_rows,), jnp.float32),
        mesh=mesh,
        scratch_types=[
            pltpu.VMEM((batch, seq_len), jnp.float32),
            pltpu.VMEM((2 * (seq_len + 4 * L),), jnp.float32),
            pltpu.VMEM((rows_per_w,), jnp.float32),
            pltpu.SemaphoreType.DMA,
        ],
        compiler_params=pltpu.CompilerParams(needs_layout_passes=False),
    )
    def body(scores_hbm, out_hbm, rowbuf, survbuf, threshbuf, sem):
        wid = lax.axis_index("s") * NC + lax.axis_index("c")
        row0 = wid * rows_per_w
        lane0 = lax.iota(jnp.int32, L) == 0
        ninf = jnp.full((L,), NEG_INF, jnp.float32)

        def filt(r, arena):
            m_even = rowbuf[r, pl.ds(0, L)]
            m_odd = rowbuf[r, pl.ds(L, L)]
            for i in range(2, n_vregs, 2):
                m_even = jnp.maximum(m_even, rowbuf[r, pl.ds(i * L, L)])
                m_odd = jnp.maximum(m_odd, rowbuf[r, pl.ds((i + 1) * L, L)])
            lb = -jnp.max(-jnp.minimum(m_even, m_odd))
            lb_v = jnp.full((L,), lb, jnp.float32)

            cnts = [0] * nseg
            for i in range(vps):
                for g in range(nseg):
                    v = rowbuf[r, pl.ds((g * vps + i) * L, L)]
                    mask = v >= lb_v
                    plsc.store_compressed(
                        survbuf.at[pl.ds(arena + g * segcap + cnts[g], L)], v, mask=mask)
                    cnts[g] = cnts[g] + plsc.all_reduce_population_count(mask)[0]
            for g in range(nseg):
                survbuf[pl.ds(arena + g * segcap + cnts[g], L)] = ninf
            return cnts

        def select(arena, cnts, prow):
            def mk_merge(base):
                def merge(i, carry):
                    thi, tlo = carry
                    bs = jnp.sort(survbuf[pl.ds(base + i * L, L)])
                    x = jnp.sort(jnp.maximum(tlo, lax.rev(bs, (0,))))
                    rx = lax.rev(x, (0,))
                    return jnp.sort(jnp.maximum(thi, rx)), jnp.sort(jnp.minimum(thi, rx))
                return merge

            carry = (ninf, ninf)
            for g in range(nseg):
                nv = (cnts[g] + L - 1) // L
                carry = lax.fori_loop(0, nv, mk_merge(arena + g * segcap), carry)
            t = -jnp.max(-carry[1])
            prow_v = jnp.full((L,), prow, jnp.int32)
            plsc.store_scatter(
                threshbuf,
                [jnp.maximum(prow_v, 0)],
                jnp.full((L,), t, jnp.float32),
                mask=lane0 & (prow_v >= 0),
            )

        def do_batch(b, carry):
            pltpu.async_copy(
                scores_hbm.at[pl.ds(row0 + b * batch, batch), :], rowbuf, sem
            ).wait()

            def do_row(r, carry):
                pc, prow = carry
                cnts = filt(r, (r % 2) * survsz)
                select((1 - r % 2) * survsz, pc, prow)
                return tuple(cnts), b * batch + r

            return lax.fori_loop(0, batch, do_row, carry)

        zero = jnp.zeros((), jnp.int32)
        pc, prow = lax.fori_loop(
            0, n_batches, do_batch, ((zero,) * nseg, -jnp.ones((), jnp.int32)))
        select(survsz, pc, prow)
        pltpu.sync_copy(threshbuf, out_hbm.at[pl.ds(row0, rows_per_w)])

    return body(scores)


def _finish_body(qs_ref, ks_ref, vs_ref, t0_ref, t1_ref, o_ref, *, seq_len, bq, hpp):
    q = qs_ref[...]
    k = ks_ref[...]
    v = vs_ref[...]
    scale = 1.0 / math.sqrt(D_K)
    s = jnp.concatenate([
        lax.dot_general(k[:, j * D_K:(j + 1) * D_K], q[:, j * D_K:(j + 1) * D_K],
                        (((1,), (1,)), ((), ())),
                        preferred_element_type=jnp.float32)
        for j in range(hpp)
    ], axis=1) * scale

    t = jnp.concatenate([t0_ref[...].reshape(1, bq), t1_ref[...].reshape(1, bq)],
                        axis=1)
    w = jnp.where(s >= t, jnp.exp(s) - 1.0, 0.0)
    denom = jnp.sum(w, axis=0) + (float(seq_len) + 1e-8)
    outs = []
    for j in range(hpp):
        vj = v[:, j * D_V:(j + 1) * D_V]
        wj = w[:, j * bq:(j + 1) * bq]
        colsum = jnp.sum(vj, axis=0)
        num = lax.dot_general(wj, vj, (((0,), (0,)), ((), ())),
                              preferred_element_type=jnp.float32)
        outs.append((num + colsum[None, :]) / denom[j * bq:(j + 1) * bq, None])
    o_ref[...] = jnp.concatenate(outs, axis=1)


def kernel(Q, K, V, W_Q, b_Q, W_K, b_K, W_V, b_V):
    batch, seq_len, d_model = Q.shape
    d_out = W_Q.shape[1]
    n_heads = d_out // D_K
    q2 = Q.reshape(seq_len, d_model)
    k2 = K.reshape(seq_len, d_model)
    v2 = V.reshape(seq_len, d_model)

    sb = 256
    proj = pl.pallas_call(
        _proj_body,
        grid=(seq_len // sb,),
        in_specs=[
            pl.BlockSpec((sb, d_model), lambda i: (i, 0)),
            pl.BlockSpec((sb, d_model), lambda i: (i, 0)),
            pl.BlockSpec((sb, d_model), lambda i: (i, 0)),
            pl.BlockSpec((d_model, d_out), lambda i: (0, 0)),
            pl.BlockSpec((1, d_out), lambda i: (0, 0)),
            pl.BlockSpec((d_model, d_out), lambda i: (0, 0)),
            pl.BlockSpec((1, d_out), lambda i: (0, 0)),
            pl.BlockSpec((d_model, d_out), lambda i: (0, 0)),
            pl.BlockSpec((1, d_out), lambda i: (0, 0)),
        ],
        out_specs=[
            pl.BlockSpec((sb, d_out), lambda i: (i, 0)),
            pl.BlockSpec((sb, d_out), lambda i: (i, 0)),
            pl.BlockSpec((sb, d_out), lambda i: (i, 0)),
        ],
        out_shape=[jax.ShapeDtypeStruct((seq_len, d_out), jnp.float32)] * 3,
    )
    qs, ks, vs = proj(q2, k2, v2,
                      W_Q, b_Q.reshape(1, d_out),
                      W_K, b_K.reshape(1, d_out),
                      W_V, b_V.reshape(1, d_out))

    bq = 128
    hpp = 2
    nqb = seq_len // bq
    scores = pl.pallas_call(
        functools.partial(_scores_body, bq=bq, hpp=hpp),
        grid=(n_heads // hpp, nqb),
        in_specs=[
            pl.BlockSpec((bq, hpp * D_K), lambda h, i: (i, h)),
            pl.BlockSpec((seq_len, hpp * D_K), lambda h, i: (0, h)),
        ],
        out_specs=pl.BlockSpec((hpp, 1, bq, seq_len), lambda h, i: (h, i, 0, 0)),
        out_shape=jax.ShapeDtypeStruct((n_heads, nqb, bq, seq_len), jnp.float32),
    )(qs, ks)

    n_rows = n_heads * seq_len
    thresh = _sc_threshold(scores.reshape(n_rows, seq_len), seq_len, n_rows)
    thresh3 = thresh.reshape(n_heads * nqb, 1, bq)

    finish = pl.pallas_call(
        functools.partial(_finish_body, seq_len=seq_len, bq=bq, hpp=hpp),
        grid=(n_heads // hpp, nqb),
        in_specs=[
            pl.BlockSpec((bq, hpp * D_K), lambda h, i: (i, h)),
            pl.BlockSpec((seq_len, hpp * D_K), lambda h, i: (0, h)),
            pl.BlockSpec((seq_len, hpp * D_V), lambda h, i: (0, h)),
            pl.BlockSpec((1, 1, bq), lambda h, i, n=nqb: (2 * h * n + i, 0, 0)),
            pl.BlockSpec((1, 1, bq), lambda h, i, n=nqb: ((2 * h + 1) * n + i, 0, 0)),
        ],
        out_specs=pl.BlockSpec((bq, hpp * D_V), lambda h, i: (i, h)),
        out_shape=jax.ShapeDtypeStruct((seq_len, d_out), jnp.float32),
    )(qs, ks, vs, thresh3, thresh3)
    return finish.reshape(batch, seq_len, d_out)


# speedup vs baseline: 1.2354x; 1.0608x over previous
import functools
import math

import jax
import jax.numpy as jnp
from jax import lax
from jax.experimental import pallas as pl
from jax.experimental.pallas import tpu as pltpu, tpu_sc as plsc

N_HEADS = 16
D_K = 64
D_V = 64
TOP_K = 32
NEG_INF = float("-inf")


def _proj_body(q_ref, k_ref, v_ref, wq_ref, bq_ref, wk_ref, bk_ref, wv_ref, bv_ref,
               qs_ref, ks_ref, vs_ref):
    qs_ref[...] = jnp.dot(q_ref[...], wq_ref[...], preferred_element_type=jnp.float32) + bq_ref[...]
    ks_ref[...] = jnp.dot(k_ref[...], wk_ref[...], preferred_element_type=jnp.float32) + bk_ref[...]
    vs_ref[...] = jnp.dot(v_ref[...], wv_ref[...], preferred_element_type=jnp.float32) + bv_ref[...]


def _scores_body(qs_ref, ks_ref, o_ref, *, bq, hpp):
    q = qs_ref[...]
    k = ks_ref[...]
    scale = 1.0 / math.sqrt(D_K)
    outs = []
    for j in range(hpp):
        s = lax.dot_general(q[:, j * D_K:(j + 1) * D_K], k[:, j * D_K:(j + 1) * D_K],
                            (((1,), (1,)), ((), ())),
                            preferred_element_type=jnp.float32) * scale
        outs.append(s[None, None])
    o_ref[...] = jnp.concatenate(outs, axis=0)


def _sc_threshold(scores, seq_len, n_rows):
    info = plsc.get_sparse_core_info()
    NC, NS, L = info.num_cores, info.num_subcores, info.num_lanes
    NW = NC * NS
    rows_per_w = n_rows // NW
    batch = 16
    n_batches = rows_per_w // batch
    n_vregs = seq_len // L

    mesh = plsc.VectorSubcoreMesh(core_axis_name="c", subcore_axis_name="s")

    nseg = 4
    vps = n_vregs // nseg
    segcap = vps * L + L
    survsz = nseg * segcap

    @functools.partial(
        pl.kernel,
        out_type=jax.ShapeDtypeStruct((n_rows,), jnp.float32),
        mesh=mesh,
        scratch_types=[
            pltpu.VMEM((2, batch, seq_len), jnp.float32),
            pltpu.VMEM((2 * (seq_len + 4 * L),), jnp.float32),
            pltpu.VMEM((rows_per_w,), jnp.float32),
            pltpu.SemaphoreType.DMA,
            pltpu.SemaphoreType.DMA,
        ],
        compiler_params=pltpu.CompilerParams(needs_layout_passes=False),
    )
    def body(scores_hbm, out_hbm, rowbuf, survbuf, threshbuf, sem0, sem1):
        wid = lax.axis_index("s") * NC + lax.axis_index("c")
        row0 = wid * rows_per_w
        lane0 = lax.iota(jnp.int32, L) == 0
        ninf = jnp.full((L,), NEG_INF, jnp.float32)

        def filt(par, r, arena):
            m_even = rowbuf[par, r, pl.ds(0, L)]
            m_odd = rowbuf[par, r, pl.ds(L, L)]
            for i in range(2, n_vregs, 2):
                m_even = jnp.maximum(m_even, rowbuf[par, r, pl.ds(i * L, L)])
                m_odd = jnp.maximum(m_odd, rowbuf[par, r, pl.ds((i + 1) * L, L)])
            lb = -jnp.max(-jnp.minimum(m_even, m_odd))
            lb_v = jnp.full((L,), lb, jnp.float32)

            cnts = [0] * nseg
            for i in range(vps):
                for g in range(nseg):
                    v = rowbuf[par, r, pl.ds((g * vps + i) * L, L)]
                    mask = v >= lb_v
                    plsc.store_compressed(
                        survbuf.at[pl.ds(arena + g * segcap + cnts[g], L)], v, mask=mask)
                    cnts[g] = cnts[g] + plsc.all_reduce_population_count(mask)[0]
            for g in range(nseg):
                survbuf[pl.ds(arena + g * segcap + cnts[g], L)] = ninf
            return cnts

        def select(arena, cnts, prow):
            def mk_merge(base):
                def merge(i, carry):
                    thi, tlo = carry
                    bs = jnp.sort(survbuf[pl.ds(base + i * L, L)])
                    x = jnp.sort(jnp.maximum(tlo, lax.rev(bs, (0,))))
                    rx = lax.rev(x, (0,))
                    return jnp.sort(jnp.maximum(thi, rx)), jnp.sort(jnp.minimum(thi, rx))
                return merge

            carry = (ninf, ninf)
            for g in range(nseg):
                nv = (cnts[g] + L - 1) // L
                carry = lax.fori_loop(0, nv, mk_merge(arena + g * segcap), carry)
            t = -jnp.max(-carry[1])
            prow_v = jnp.full((L,), prow, jnp.int32)
            plsc.store_scatter(
                threshbuf,
                [jnp.maximum(prow_v, 0)],
                jnp.full((L,), t, jnp.float32),
                mask=lane0 & (prow_v >= 0),
            )

        def issue(b, p):
            pltpu.async_copy(
                scores_hbm.at[pl.ds(row0 + b * batch, batch), :], rowbuf.at[p],
                sem0 if p == 0 else sem1)

        def drain(p):
            pltpu.make_async_copy(
                scores_hbm.at[pl.ds(row0, batch), :], rowbuf.at[p],
                sem0 if p == 0 else sem1).wait()

        def do_batch(b, carry):
            par = b % 2
            even = par == 0
            @pl.when(even)
            def _():
                drain(0)
            @pl.when(~even)
            def _():
                drain(1)
            @pl.when(even & (b + 1 < n_batches))
            def _():
                issue(b + 1, 1)
            @pl.when((~even) & (b + 1 < n_batches))
            def _():
                issue(b + 1, 0)

            def do_row(r, carry):
                pc, prow = carry
                cnts = filt(par, r, (r % 2) * survsz)
                select((1 - r % 2) * survsz, pc, prow)
                return tuple(cnts), b * batch + r

            return lax.fori_loop(0, batch, do_row, carry)

        issue(0, 0)
        zero = jnp.zeros((), jnp.int32)
        pc, prow = lax.fori_loop(
            0, n_batches, do_batch, ((zero,) * nseg, -jnp.ones((), jnp.int32)))
        select(survsz, pc, prow)
        pltpu.sync_copy(threshbuf, out_hbm.at[pl.ds(row0, rows_per_w)])

    return body(scores)


def _finish_body(qs_ref, ks_ref, vs_ref, t0_ref, t1_ref, o_ref, *, seq_len, bq, hpp):
    q = qs_ref[...]
    k = ks_ref[...]
    v = vs_ref[...]
    scale = 1.0 / math.sqrt(D_K)
    s = jnp.concatenate([
        lax.dot_general(k[:, j * D_K:(j + 1) * D_K], q[:, j * D_K:(j + 1) * D_K],
                        (((1,), (1,)), ((), ())),
                        preferred_element_type=jnp.float32)
        for j in range(hpp)
    ], axis=1) * scale

    t = jnp.concatenate([t0_ref[...].reshape(1, bq), t1_ref[...].reshape(1, bq)],
                        axis=1)
    w = jnp.where(s >= t, jnp.exp(s) - 1.0, 0.0)
    denom = jnp.sum(w, axis=0) + (float(seq_len) + 1e-8)
    outs = []
    for j in range(hpp):
        vj = v[:, j * D_V:(j + 1) * D_V]
        wj = w[:, j * bq:(j + 1) * bq]
        colsum = jnp.sum(vj, axis=0)
        num = lax.dot_general(wj, vj, (((0,), (0,)), ((), ())),
                              preferred_element_type=jnp.float32)
        outs.append((num + colsum[None, :]) / denom[j * bq:(j + 1) * bq, None])
    o_ref[...] = jnp.concatenate(outs, axis=1)


def kernel(Q, K, V, W_Q, b_Q, W_K, b_K, W_V, b_V):
    batch, seq_len, d_model = Q.shape
    d_out = W_Q.shape[1]
    n_heads = d_out // D_K
    q2 = Q.reshape(seq_len, d_model)
    k2 = K.reshape(seq_len, d_model)
    v2 = V.reshape(seq_len, d_model)

    sb = 256
    proj = pl.pallas_call(
        _proj_body,
        grid=(seq_len // sb,),
        in_specs=[
            pl.BlockSpec((sb, d_model), lambda i: (i, 0)),
            pl.BlockSpec((sb, d_model), lambda i: (i, 0)),
            pl.BlockSpec((sb, d_model), lambda i: (i, 0)),
            pl.BlockSpec((d_model, d_out), lambda i: (0, 0)),
            pl.BlockSpec((1, d_out), lambda i: (0, 0)),
            pl.BlockSpec((d_model, d_out), lambda i: (0, 0)),
            pl.BlockSpec((1, d_out), lambda i: (0, 0)),
            pl.BlockSpec((d_model, d_out), lambda i: (0, 0)),
            pl.BlockSpec((1, d_out), lambda i: (0, 0)),
        ],
        out_specs=[
            pl.BlockSpec((sb, d_out), lambda i: (i, 0)),
            pl.BlockSpec((sb, d_out), lambda i: (i, 0)),
            pl.BlockSpec((sb, d_out), lambda i: (i, 0)),
        ],
        out_shape=[jax.ShapeDtypeStruct((seq_len, d_out), jnp.float32)] * 3,
    )
    qs, ks, vs = proj(q2, k2, v2,
                      W_Q, b_Q.reshape(1, d_out),
                      W_K, b_K.reshape(1, d_out),
                      W_V, b_V.reshape(1, d_out))

    bq = 128
    hpp = 2
    nqb = seq_len // bq
    scores = pl.pallas_call(
        functools.partial(_scores_body, bq=bq, hpp=hpp),
        grid=(n_heads // hpp, nqb),
        in_specs=[
            pl.BlockSpec((bq, hpp * D_K), lambda h, i: (i, h)),
            pl.BlockSpec((seq_len, hpp * D_K), lambda h, i: (0, h)),
        ],
        out_specs=pl.BlockSpec((hpp, 1, bq, seq_len), lambda h, i: (h, i, 0, 0)),
        out_shape=jax.ShapeDtypeStruct((n_heads, nqb, bq, seq_len), jnp.float32),
    )(qs, ks)

    n_rows = n_heads * seq_len
    thresh = _sc_threshold(scores.reshape(n_rows, seq_len), seq_len, n_rows)
    thresh3 = thresh.reshape(n_heads * nqb, 1, bq)

    finish = pl.pallas_call(
        functools.partial(_finish_body, seq_len=seq_len, bq=bq, hpp=hpp),
        grid=(n_heads // hpp, nqb),
        in_specs=[
            pl.BlockSpec((bq, hpp * D_K), lambda h, i: (i, h)),
            pl.BlockSpec((seq_len, hpp * D_K), lambda h, i: (0, h)),
            pl.BlockSpec((seq_len, hpp * D_V), lambda h, i: (0, h)),
            pl.BlockSpec((1, 1, bq), lambda h, i, n=nqb: (2 * h * n + i, 0, 0)),
            pl.BlockSpec((1, 1, bq), lambda h, i, n=nqb: ((2 * h + 1) * n + i, 0, 0)),
        ],
        out_specs=pl.BlockSpec((bq, hpp * D_V), lambda h, i: (i, h)),
        out_shape=jax.ShapeDtypeStruct((seq_len, d_out), jnp.float32),
    )(qs, ks, vs, thresh3, thresh3)
    return finish.reshape(batch, seq_len, d_out)


# speedup vs baseline: 1.2758x; 1.0326x over previous
import functools
import math

import jax
import jax.numpy as jnp
from jax import lax
from jax.experimental import pallas as pl
from jax.experimental.pallas import tpu as pltpu, tpu_sc as plsc

N_HEADS = 16
D_K = 64
D_V = 64
TOP_K = 32
NEG_INF = float("-inf")


def _proj_body(q_ref, k_ref, v_ref, wq_ref, bq_ref, wk_ref, bk_ref, wv_ref, bv_ref,
               qs_ref, ks_ref, vs_ref):
    qs_ref[...] = jnp.dot(q_ref[...], wq_ref[...], preferred_element_type=jnp.float32) + bq_ref[...]
    ks_ref[...] = jnp.dot(k_ref[...], wk_ref[...], preferred_element_type=jnp.float32) + bk_ref[...]
    vs_ref[...] = jnp.dot(v_ref[...], wv_ref[...], preferred_element_type=jnp.float32) + bv_ref[...]


def _scores_body(qs_ref, ks_ref, o_ref, *, bq, hpp):
    q = qs_ref[...]
    k = ks_ref[...]
    scale = 1.0 / math.sqrt(D_K)
    outs = []
    for j in range(hpp):
        s = lax.dot_general(q[:, j * D_K:(j + 1) * D_K], k[:, j * D_K:(j + 1) * D_K],
                            (((1,), (1,)), ((), ())),
                            preferred_element_type=jnp.float32) * scale
        outs.append(s[None, None])
    o_ref[...] = jnp.concatenate(outs, axis=0)


def _sc_threshold(scores, seq_len, n_rows):
    info = plsc.get_sparse_core_info()
    NC, NS, L = info.num_cores, info.num_subcores, info.num_lanes
    NW = NC * NS
    rows_per_w = n_rows // NW
    batch = 16
    n_batches = rows_per_w // batch
    n_vregs = seq_len // L

    mesh = plsc.VectorSubcoreMesh(core_axis_name="c", subcore_axis_name="s")

    nseg = 4
    vps = n_vregs // nseg
    segcap = vps * L + L
    survsz = nseg * segcap

    @functools.partial(
        pl.kernel,
        out_type=jax.ShapeDtypeStruct((n_rows,), jnp.float32),
        mesh=mesh,
        scratch_types=[
            pltpu.VMEM((2, batch, seq_len), jnp.float32),
            pltpu.VMEM((2 * (seq_len + 4 * L),), jnp.float32),
            pltpu.VMEM((rows_per_w,), jnp.float32),
            pltpu.SemaphoreType.DMA,
            pltpu.SemaphoreType.DMA,
        ],
        compiler_params=pltpu.CompilerParams(needs_layout_passes=False),
    )
    def body(scores_hbm, out_hbm, rowbuf, survbuf, threshbuf, sem0, sem1):
        wid = lax.axis_index("s") * NC + lax.axis_index("c")
        row0 = wid * rows_per_w
        lane0 = lax.iota(jnp.int32, L) == 0
        ninf = jnp.full((L,), NEG_INF, jnp.float32)

        def filt(par, r, arena):
            m_even = rowbuf[par, r, pl.ds(0, L)]
            m_odd = rowbuf[par, r, pl.ds(L, L)]
            for i in range(2, n_vregs, 2):
                m_even = jnp.maximum(m_even, rowbuf[par, r, pl.ds(i * L, L)])
                m_odd = jnp.maximum(m_odd, rowbuf[par, r, pl.ds((i + 1) * L, L)])
            lb = -jnp.max(-jnp.minimum(m_even, m_odd))
            lb_v = jnp.full((L,), lb, jnp.float32)

            cnts = [0] * nseg
            for i in range(vps):
                for g in range(nseg):
                    v = rowbuf[par, r, pl.ds((g * vps + i) * L, L)]
                    mask = v >= lb_v
                    plsc.store_compressed(
                        survbuf.at[pl.ds(arena + g * segcap + cnts[g], L)], v, mask=mask)
                    cnts[g] = cnts[g] + plsc.all_reduce_population_count(mask)[0]
            for g in range(nseg):
                survbuf[pl.ds(arena + g * segcap + cnts[g], L)] = ninf
            return cnts

        def select(arena, cnts, prow):
            def mk_merge(base):
                def merge(i, carry):
                    thi, tlo = carry
                    bs = jnp.sort(survbuf[pl.ds(base + i * L, L)])
                    x = jnp.sort(jnp.maximum(tlo, lax.rev(bs, (0,))))
                    rx = lax.rev(x, (0,))
                    return jnp.sort(jnp.maximum(thi, rx)), jnp.sort(jnp.minimum(thi, rx))
                return merge

            carry = (ninf, ninf)
            for g in range(nseg):
                nv = (cnts[g] + L - 1) // L
                carry = lax.fori_loop(0, nv, mk_merge(arena + g * segcap), carry)
            t = -jnp.max(-carry[1])
            prow_v = jnp.full((L,), prow, jnp.int32)
            plsc.store_scatter(
                threshbuf,
                [jnp.maximum(prow_v, 0)],
                jnp.full((L,), t, jnp.float32),
                mask=lane0 & (prow_v >= 0),
            )

        def issue(b, p):
            pltpu.async_copy(
                scores_hbm.at[pl.ds(row0 + b * batch, batch), :], rowbuf.at[p],
                sem0 if p == 0 else sem1)

        def drain(p):
            pltpu.make_async_copy(
                scores_hbm.at[pl.ds(row0, batch), :], rowbuf.at[p],
                sem0 if p == 0 else sem1).wait()

        def do_batch(b, carry):
            par = b % 2
            even = par == 0
            @pl.when(even)
            def _():
                drain(0)
            @pl.when(~even)
            def _():
                drain(1)
            @pl.when(even & (b + 1 < n_batches))
            def _():
                issue(b + 1, 1)
            @pl.when((~even) & (b + 1 < n_batches))
            def _():
                issue(b + 1, 0)

            def do_row(r, carry):
                pc, prow = carry
                cnts = filt(par, r, (r % 2) * survsz)
                select((1 - r % 2) * survsz, pc, prow)
                return tuple(cnts), b * batch + r

            return lax.fori_loop(0, batch, do_row, carry)

        issue(0, 0)
        zero = jnp.zeros((), jnp.int32)
        pc, prow = lax.fori_loop(
            0, n_batches, do_batch, ((zero,) * nseg, -jnp.ones((), jnp.int32)))
        select(survsz, pc, prow)
        pltpu.sync_copy(threshbuf, out_hbm.at[pl.ds(row0, rows_per_w)])

    return body(scores)


def _finish_body(qs_ref, ks_ref, vs_ref, t0_ref, t1_ref, o_ref, *, seq_len, bq, hpp):
    q = qs_ref[...]
    k = ks_ref[...]
    v = vs_ref[...]
    scale = 1.0 / math.sqrt(D_K)
    s = jnp.concatenate([
        lax.dot_general(k[:, j * D_K:(j + 1) * D_K], q[:, j * D_K:(j + 1) * D_K],
                        (((1,), (1,)), ((), ())),
                        preferred_element_type=jnp.float32)
        for j in range(hpp)
    ], axis=1) * scale

    t = jnp.concatenate([t0_ref[...].reshape(1, bq), t1_ref[...].reshape(1, bq)],
                        axis=1)
    w = jnp.where(s >= t, jnp.exp(s) - 1.0, 0.0)
    denom = jnp.sum(w, axis=0) + (float(seq_len) + 1e-8)
    outs = []
    for j in range(hpp):
        vj = v[:, j * D_V:(j + 1) * D_V]
        wj = w[:, j * bq:(j + 1) * bq]
        colsum = jnp.sum(vj, axis=0)
        num = lax.dot_general(wj, vj, (((0,), (0,)), ((), ())),
                              preferred_element_type=jnp.float32)
        outs.append((num + colsum[None, :]) / denom[j * bq:(j + 1) * bq, None])
    o_ref[...] = jnp.concatenate(outs, axis=1)


def kernel(Q, K, V, W_Q, b_Q, W_K, b_K, W_V, b_V):
    batch, seq_len, d_model = Q.shape
    d_out = W_Q.shape[1]
    n_heads = d_out // D_K
    q2 = Q.reshape(seq_len, d_model)
    k2 = K.reshape(seq_len, d_model)
    v2 = V.reshape(seq_len, d_model)

    sb = 256
    proj = pl.pallas_call(
        _proj_body,
        grid=(seq_len // sb,),
        in_specs=[
            pl.BlockSpec((sb, d_model), lambda i: (i, 0)),
            pl.BlockSpec((sb, d_model), lambda i: (i, 0)),
            pl.BlockSpec((sb, d_model), lambda i: (i, 0)),
            pl.BlockSpec((d_model, d_out), lambda i: (0, 0)),
            pl.BlockSpec((1, d_out), lambda i: (0, 0)),
            pl.BlockSpec((d_model, d_out), lambda i: (0, 0)),
            pl.BlockSpec((1, d_out), lambda i: (0, 0)),
            pl.BlockSpec((d_model, d_out), lambda i: (0, 0)),
            pl.BlockSpec((1, d_out), lambda i: (0, 0)),
        ],
        out_specs=[
            pl.BlockSpec((sb, d_out), lambda i: (i, 0)),
            pl.BlockSpec((sb, d_out), lambda i: (i, 0)),
            pl.BlockSpec((sb, d_out), lambda i: (i, 0)),
        ],
        out_shape=[jax.ShapeDtypeStruct((seq_len, d_out), jnp.float32)] * 3,
    )
    qs, ks, vs = proj(q2, k2, v2,
                      W_Q, b_Q.reshape(1, d_out),
                      W_K, b_K.reshape(1, d_out),
                      W_V, b_V.reshape(1, d_out))

    bq = 128
    hpp = 2
    nqb = seq_len // bq
    nh2 = n_heads // 2
    threshes = []
    for half in range(2):
        off = half * (nh2 // hpp)
        sc_half = pl.pallas_call(
            functools.partial(_scores_body, bq=bq, hpp=hpp),
            grid=(nh2 // hpp, nqb),
            in_specs=[
                pl.BlockSpec((bq, hpp * D_K), lambda h, i, o=off: (i, h + o)),
                pl.BlockSpec((seq_len, hpp * D_K), lambda h, i, o=off: (0, h + o)),
            ],
            out_specs=pl.BlockSpec((hpp, 1, bq, seq_len), lambda h, i: (h, i, 0, 0)),
            out_shape=jax.ShapeDtypeStruct((nh2, nqb, bq, seq_len), jnp.float32),
        )(qs, ks)
        n_rows = nh2 * seq_len
        threshes.append(_sc_threshold(sc_half.reshape(n_rows, seq_len), seq_len, n_rows))
    thresh3 = jnp.concatenate(threshes).reshape(n_heads * nqb, 1, bq)

    finish = pl.pallas_call(
        functools.partial(_finish_body, seq_len=seq_len, bq=bq, hpp=hpp),
        grid=(n_heads // hpp, nqb),
        in_specs=[
            pl.BlockSpec((bq, hpp * D_K), lambda h, i: (i, h)),
            pl.BlockSpec((seq_len, hpp * D_K), lambda h, i: (0, h)),
            pl.BlockSpec((seq_len, hpp * D_V), lambda h, i: (0, h)),
            pl.BlockSpec((1, 1, bq), lambda h, i, n=nqb: (2 * h * n + i, 0, 0)),
            pl.BlockSpec((1, 1, bq), lambda h, i, n=nqb: ((2 * h + 1) * n + i, 0, 0)),
        ],
        out_specs=pl.BlockSpec((bq, hpp * D_V), lambda h, i: (i, h)),
        out_shape=jax.ShapeDtypeStruct((seq_len, d_out), jnp.float32),
    )(qs, ks, vs, thresh3, thresh3)
    return finish.reshape(batch, seq_len, d_out)


# speedup vs baseline: 1.3532x; 1.0607x over previous
import functools
import math

import jax
import jax.numpy as jnp
from jax import lax
from jax.experimental import pallas as pl
from jax.experimental.pallas import tpu as pltpu, tpu_sc as plsc

N_HEADS = 16
D_K = 64
D_V = 64
TOP_K = 32
NEG_INF = float("-inf")


def _proj_body(q_ref, k_ref, v_ref, wq_ref, bq_ref, wk_ref, bk_ref, wv_ref, bv_ref,
               qs_ref, ks_ref, vs_ref):
    qs_ref[...] = jnp.dot(q_ref[...], wq_ref[...], preferred_element_type=jnp.float32) + bq_ref[...]
    ks_ref[...] = jnp.dot(k_ref[...], wk_ref[...], preferred_element_type=jnp.float32) + bk_ref[...]
    vs_ref[...] = jnp.dot(v_ref[...], wv_ref[...], preferred_element_type=jnp.float32) + bv_ref[...]


def _scores_body(qs_ref, ks_ref, o_ref, *, bq, hpp):
    q = qs_ref[...]
    k = ks_ref[...]
    scale = 1.0 / math.sqrt(D_K)
    outs = []
    for j in range(hpp):
        s = lax.dot_general(q[:, j * D_K:(j + 1) * D_K], k[:, j * D_K:(j + 1) * D_K],
                            (((1,), (1,)), ((), ())),
                            preferred_element_type=jnp.float32) * scale
        outs.append(s[None, None])
    o_ref[...] = jnp.concatenate(outs, axis=0)


def _sc_threshold(scores, seq_len, n_rows):
    info = plsc.get_sparse_core_info()
    NC, NS, L = info.num_cores, info.num_subcores, info.num_lanes
    NW = NC * NS
    rows_per_w = n_rows // NW
    batch = 16
    n_batches = rows_per_w // batch
    n_vregs = seq_len // L

    mesh = plsc.VectorSubcoreMesh(core_axis_name="c", subcore_axis_name="s")

    nseg = 4
    vps = n_vregs // nseg
    segcap = vps * L + L
    survsz = nseg * segcap

    @functools.partial(
        pl.kernel,
        out_type=jax.ShapeDtypeStruct((n_rows,), jnp.float32),
        mesh=mesh,
        scratch_types=[
            pltpu.VMEM((2, batch, seq_len), jnp.float32),
            pltpu.VMEM((2 * (seq_len + 4 * L),), jnp.float32),
            pltpu.VMEM((rows_per_w,), jnp.float32),
            pltpu.SemaphoreType.DMA,
            pltpu.SemaphoreType.DMA,
        ],
        compiler_params=pltpu.CompilerParams(needs_layout_passes=False),
    )
    def body(scores_hbm, out_hbm, rowbuf, survbuf, threshbuf, sem0, sem1):
        wid = lax.axis_index("s") * NC + lax.axis_index("c")
        row0 = wid * rows_per_w
        lane0 = lax.iota(jnp.int32, L) == 0
        ninf = jnp.full((L,), NEG_INF, jnp.float32)

        def filt(par, r, arena):
            m_even = rowbuf[par, r, pl.ds(0, L)]
            m_odd = rowbuf[par, r, pl.ds(L, L)]
            for i in range(2, n_vregs, 2):
                m_even = jnp.maximum(m_even, rowbuf[par, r, pl.ds(i * L, L)])
                m_odd = jnp.maximum(m_odd, rowbuf[par, r, pl.ds((i + 1) * L, L)])
            lb = -jnp.max(-jnp.minimum(m_even, m_odd))
            lb_v = jnp.full((L,), lb, jnp.float32)

            cnts = [0] * nseg
            for i in range(vps):
                for g in range(nseg):
                    v = rowbuf[par, r, pl.ds((g * vps + i) * L, L)]
                    mask = v >= lb_v
                    plsc.store_compressed(
                        survbuf.at[pl.ds(arena + g * segcap + cnts[g], L)], v, mask=mask)
                    cnts[g] = cnts[g] + plsc.all_reduce_population_count(mask)[0]
            for g in range(nseg):
                survbuf[pl.ds(arena + g * segcap + cnts[g], L)] = ninf
            return cnts

        def select(arena, cnts, prow):
            def mk_merge(base):
                def merge(i, carry):
                    thi, tlo = carry
                    bs = jnp.sort(survbuf[pl.ds(base + i * L, L)])
                    x = jnp.sort(jnp.maximum(tlo, lax.rev(bs, (0,))))
                    rx = lax.rev(x, (0,))
                    return jnp.sort(jnp.maximum(thi, rx)), jnp.sort(jnp.minimum(thi, rx))
                return merge

            carry = (ninf, ninf)
            for g in range(nseg):
                nv = (cnts[g] + L - 1) // L
                carry = lax.fori_loop(0, nv, mk_merge(arena + g * segcap), carry)
            t = -jnp.max(-carry[1])
            prow_v = jnp.full((L,), prow, jnp.int32)
            plsc.store_scatter(
                threshbuf,
                [jnp.maximum(prow_v, 0)],
                jnp.full((L,), t, jnp.float32),
                mask=lane0 & (prow_v >= 0),
            )

        def issue(b, p):
            pltpu.async_copy(
                scores_hbm.at[pl.ds(row0 + b * batch, batch), :], rowbuf.at[p],
                sem0 if p == 0 else sem1)

        def drain(p):
            pltpu.make_async_copy(
                scores_hbm.at[pl.ds(row0, batch), :], rowbuf.at[p],
                sem0 if p == 0 else sem1).wait()

        def do_batch(b, carry):
            par = b % 2
            even = par == 0
            @pl.when(even)
            def _():
                drain(0)
            @pl.when(~even)
            def _():
                drain(1)
            @pl.when(even & (b + 1 < n_batches))
            def _():
                issue(b + 1, 1)
            @pl.when((~even) & (b + 1 < n_batches))
            def _():
                issue(b + 1, 0)

            def do_row(r, carry):
                pc, prow = carry
                cnts = filt(par, r, (r % 2) * survsz)
                select((1 - r % 2) * survsz, pc, prow)
                return tuple(cnts), b * batch + r

            return lax.fori_loop(0, batch, do_row, carry)

        issue(0, 0)
        zero = jnp.zeros((), jnp.int32)
        pc, prow = lax.fori_loop(
            0, n_batches, do_batch, ((zero,) * nseg, -jnp.ones((), jnp.int32)))
        select(survsz, pc, prow)
        pltpu.sync_copy(threshbuf, out_hbm.at[pl.ds(row0, rows_per_w)])

    return body(scores)


def _finish_body(qs_ref, ks_ref, vs_ref, t0_ref, t1_ref, o_ref, *, seq_len, bq, hpp):
    q = qs_ref[...]
    k = ks_ref[...]
    v = vs_ref[...]
    scale = 1.0 / math.sqrt(D_K)
    s = jnp.concatenate([
        lax.dot_general(k[:, j * D_K:(j + 1) * D_K], q[:, j * D_K:(j + 1) * D_K],
                        (((1,), (1,)), ((), ())),
                        preferred_element_type=jnp.float32)
        for j in range(hpp)
    ], axis=1) * scale

    t = jnp.concatenate([t0_ref[...].reshape(1, bq), t1_ref[...].reshape(1, bq)],
                        axis=1)
    w = jnp.where(s >= t, jnp.exp(s) - 1.0, 0.0)
    denom = jnp.sum(w, axis=0) + (float(seq_len) + 1e-8)
    outs = []
    for j in range(hpp):
        vj = v[:, j * D_V:(j + 1) * D_V]
        wj = w[:, j * bq:(j + 1) * bq]
        colsum = jnp.sum(vj, axis=0)
        num = lax.dot_general(wj, vj, (((0,), (0,)), ((), ())),
                              preferred_element_type=jnp.float32)
        outs.append((num + colsum[None, :]) / denom[j * bq:(j + 1) * bq, None])
    o_ref[...] = jnp.concatenate(outs, axis=1)


def kernel(Q, K, V, W_Q, b_Q, W_K, b_K, W_V, b_V):
    batch, seq_len, d_model = Q.shape
    d_out = W_Q.shape[1]
    n_heads = d_out // D_K
    q2 = Q.reshape(seq_len, d_model)
    k2 = K.reshape(seq_len, d_model)
    v2 = V.reshape(seq_len, d_model)

    sb = 256
    proj = pl.pallas_call(
        _proj_body,
        grid=(seq_len // sb,),
        in_specs=[
            pl.BlockSpec((sb, d_model), lambda i: (i, 0)),
            pl.BlockSpec((sb, d_model), lambda i: (i, 0)),
            pl.BlockSpec((sb, d_model), lambda i: (i, 0)),
            pl.BlockSpec((d_model, d_out), lambda i: (0, 0)),
            pl.BlockSpec((1, d_out), lambda i: (0, 0)),
            pl.BlockSpec((d_model, d_out), lambda i: (0, 0)),
            pl.BlockSpec((1, d_out), lambda i: (0, 0)),
            pl.BlockSpec((d_model, d_out), lambda i: (0, 0)),
            pl.BlockSpec((1, d_out), lambda i: (0, 0)),
        ],
        out_specs=[
            pl.BlockSpec((sb, d_out), lambda i: (i, 0)),
            pl.BlockSpec((sb, d_out), lambda i: (i, 0)),
            pl.BlockSpec((sb, d_out), lambda i: (i, 0)),
        ],
        out_shape=[jax.ShapeDtypeStruct((seq_len, d_out), jnp.float32)] * 3,
    )
    qs, ks, vs = proj(q2, k2, v2,
                      W_Q, b_Q.reshape(1, d_out),
                      W_K, b_K.reshape(1, d_out),
                      W_V, b_V.reshape(1, d_out))

    bq = 128
    hpp = 2
    nqb = seq_len // bq
    nh2 = n_heads // 2
    threshes = []
    for half in range(2):
        off = half * (nh2 // hpp)
        sc_half = pl.pallas_call(
            functools.partial(_scores_body, bq=bq, hpp=hpp),
            grid=(nh2 // hpp, nqb),
            in_specs=[
                pl.BlockSpec((bq, hpp * D_K), lambda h, i, o=off: (i, h + o)),
                pl.BlockSpec((seq_len, hpp * D_K), lambda h, i, o=off: (0, h + o)),
            ],
            out_specs=pl.BlockSpec((hpp, 1, bq, seq_len), lambda h, i: (h, i, 0, 0)),
            out_shape=jax.ShapeDtypeStruct((nh2, nqb, bq, seq_len), jnp.float32),
        )(qs, ks)
        n_rows = nh2 * seq_len
        threshes.append(_sc_threshold(sc_half.reshape(n_rows, seq_len), seq_len, n_rows))
    outs = []
    for half in range(2):
        off = half * (nh2 // hpp)
        t3 = threshes[half].reshape(nh2 * nqb, 1, bq)
        outs.append(pl.pallas_call(
            functools.partial(_finish_body, seq_len=seq_len, bq=bq, hpp=hpp),
            grid=(nh2 // hpp, nqb),
            in_specs=[
                pl.BlockSpec((bq, hpp * D_K), lambda h, i, o=off: (i, h + o)),
                pl.BlockSpec((seq_len, hpp * D_K), lambda h, i, o=off: (0, h + o)),
                pl.BlockSpec((seq_len, hpp * D_V), lambda h, i, o=off: (0, h + o)),
                pl.BlockSpec((1, 1, bq), lambda h, i, n=nqb: (2 * h * n + i, 0, 0)),
                pl.BlockSpec((1, 1, bq), lambda h, i, n=nqb: ((2 * h + 1) * n + i, 0, 0)),
            ],
            out_specs=pl.BlockSpec((bq, hpp * D_V), lambda h, i: (i, h)),
            out_shape=jax.ShapeDtypeStruct((seq_len, nh2 * D_V), jnp.float32),
        )(qs, ks, vs, t3, t3))
    return jnp.concatenate(outs, axis=1).reshape(batch, seq_len, d_out)


# speedup vs baseline: 1.4093x; 1.0414x over previous
import functools
import math

import jax
import jax.numpy as jnp
from jax import lax
from jax.experimental import pallas as pl
from jax.experimental.pallas import tpu as pltpu, tpu_sc as plsc

N_HEADS = 16
D_K = 64
D_V = 64
TOP_K = 32
NEG_INF = float("-inf")


def _proj_body(q_ref, k_ref, v_ref, wq_ref, bq_ref, wk_ref, bk_ref, wv_ref, bv_ref,
               qs_ref, ks_ref, vs_ref):
    qs_ref[...] = jnp.dot(q_ref[...], wq_ref[...], preferred_element_type=jnp.float32) + bq_ref[...]
    ks_ref[...] = jnp.dot(k_ref[...], wk_ref[...], preferred_element_type=jnp.float32) + bk_ref[...]
    vs_ref[...] = jnp.dot(v_ref[...], wv_ref[...], preferred_element_type=jnp.float32) + bv_ref[...]


def _scores_body(qs_ref, ks_ref, o_ref, *, bq, hpp):
    q = qs_ref[...]
    k = ks_ref[...]
    scale = 1.0 / math.sqrt(D_K)
    outs = []
    for j in range(hpp):
        s = lax.dot_general(q[:, j * D_K:(j + 1) * D_K], k[:, j * D_K:(j + 1) * D_K],
                            (((1,), (1,)), ((), ())),
                            preferred_element_type=jnp.float32) * scale
        outs.append(s[None, None])
    o_ref[...] = jnp.concatenate(outs, axis=0)


def _sc_threshold(scores, seq_len, n_rows):
    info = plsc.get_sparse_core_info()
    NC, NS, L = info.num_cores, info.num_subcores, info.num_lanes
    NW = NC * NS
    rows_per_w = n_rows // NW
    batch = 16
    n_batches = rows_per_w // batch
    n_vregs = seq_len // L

    mesh = plsc.VectorSubcoreMesh(core_axis_name="c", subcore_axis_name="s")

    nseg = 4
    vps = n_vregs // nseg
    segcap = vps * L + L
    survsz = nseg * segcap

    @functools.partial(
        pl.kernel,
        out_type=jax.ShapeDtypeStruct((n_rows,), jnp.float32),
        mesh=mesh,
        scratch_types=[
            pltpu.VMEM((2, batch, seq_len), jnp.float32),
            pltpu.VMEM((2 * (seq_len + 4 * L),), jnp.float32),
            pltpu.VMEM((rows_per_w,), jnp.float32),
            pltpu.SemaphoreType.DMA,
            pltpu.SemaphoreType.DMA,
        ],
        compiler_params=pltpu.CompilerParams(needs_layout_passes=False),
    )
    def body(scores_hbm, out_hbm, rowbuf, survbuf, threshbuf, sem0, sem1):
        wid = lax.axis_index("s") * NC + lax.axis_index("c")
        row0 = wid * rows_per_w
        lane0 = lax.iota(jnp.int32, L) == 0
        ninf = jnp.full((L,), NEG_INF, jnp.float32)

        def filt(par, r, arena):
            m_even = rowbuf[par, r, pl.ds(0, L)]
            m_odd = rowbuf[par, r, pl.ds(L, L)]
            for i in range(2, n_vregs, 2):
                m_even = jnp.maximum(m_even, rowbuf[par, r, pl.ds(i * L, L)])
                m_odd = jnp.maximum(m_odd, rowbuf[par, r, pl.ds((i + 1) * L, L)])
            lb = -jnp.max(-jnp.minimum(m_even, m_odd))
            lb_v = jnp.full((L,), lb, jnp.float32)

            cnts = [0] * nseg
            for i in range(vps):
                for g in range(nseg):
                    v = rowbuf[par, r, pl.ds((g * vps + i) * L, L)]
                    mask = v >= lb_v
                    plsc.store_compressed(
                        survbuf.at[pl.ds(arena + g * segcap + cnts[g], L)], v, mask=mask)
                    cnts[g] = cnts[g] + plsc.all_reduce_population_count(mask)[0]
            for g in range(nseg):
                survbuf[pl.ds(arena + g * segcap + cnts[g], L)] = ninf
            return cnts

        def select(arena, cnts, prow):
            def mk_merge(base):
                def merge(i, carry):
                    thi, tlo = carry
                    bs = jnp.sort(survbuf[pl.ds(base + i * L, L)])
                    x = jnp.sort(jnp.maximum(tlo, lax.rev(bs, (0,))))
                    rx = lax.rev(x, (0,))
                    return jnp.sort(jnp.maximum(thi, rx)), jnp.sort(jnp.minimum(thi, rx))
                return merge

            carry = (ninf, ninf)
            for g in range(nseg):
                nv = (cnts[g] + L - 1) // L
                carry = lax.fori_loop(0, nv, mk_merge(arena + g * segcap), carry)
            t = -jnp.max(-carry[1])
            prow_v = jnp.full((L,), prow, jnp.int32)
            plsc.store_scatter(
                threshbuf,
                [jnp.maximum(prow_v, 0)],
                jnp.full((L,), t, jnp.float32),
                mask=lane0 & (prow_v >= 0),
            )

        def issue(b, p):
            pltpu.async_copy(
                scores_hbm.at[pl.ds(row0 + b * batch, batch), :], rowbuf.at[p],
                sem0 if p == 0 else sem1)

        def drain(p):
            pltpu.make_async_copy(
                scores_hbm.at[pl.ds(row0, batch), :], rowbuf.at[p],
                sem0 if p == 0 else sem1).wait()

        def do_batch(b, carry):
            par = b % 2
            even = par == 0
            @pl.when(even)
            def _():
                drain(0)
            @pl.when(~even)
            def _():
                drain(1)
            @pl.when(even & (b + 1 < n_batches))
            def _():
                issue(b + 1, 1)
            @pl.when((~even) & (b + 1 < n_batches))
            def _():
                issue(b + 1, 0)

            def do_row(r, carry):
                pc, prow = carry
                cnts = filt(par, r, (r % 2) * survsz)
                select((1 - r % 2) * survsz, pc, prow)
                return tuple(cnts), b * batch + r

            return lax.fori_loop(0, batch, do_row, carry)

        issue(0, 0)
        zero = jnp.zeros((), jnp.int32)
        pc, prow = lax.fori_loop(
            0, n_batches, do_batch, ((zero,) * nseg, -jnp.ones((), jnp.int32)))
        select(survsz, pc, prow)
        pltpu.sync_copy(threshbuf, out_hbm.at[pl.ds(row0, rows_per_w)])

    return body(scores)


def _finish_body(qs_ref, ks_ref, vs_ref, t0_ref, t1_ref, o_ref, *, seq_len, bq, hpp):
    q = qs_ref[...]
    k = ks_ref[...]
    v = vs_ref[...]
    scale = 1.0 / math.sqrt(D_K)
    s = jnp.concatenate([
        lax.dot_general(k[:, j * D_K:(j + 1) * D_K], q[:, j * D_K:(j + 1) * D_K],
                        (((1,), (1,)), ((), ())),
                        preferred_element_type=jnp.float32)
        for j in range(hpp)
    ], axis=1) * scale

    t = jnp.concatenate([t0_ref[...].reshape(1, bq), t1_ref[...].reshape(1, bq)],
                        axis=1)
    w = jnp.where(s >= t, jnp.exp(s) - 1.0, 0.0)
    denom = jnp.sum(w, axis=0) + (float(seq_len) + 1e-8)
    outs = []
    for j in range(hpp):
        vj = v[:, j * D_V:(j + 1) * D_V]
        wj = w[:, j * bq:(j + 1) * bq]
        colsum = jnp.sum(vj, axis=0)
        num = lax.dot_general(wj, vj, (((0,), (0,)), ((), ())),
                              preferred_element_type=jnp.float32)
        outs.append((num + colsum[None, :]) / denom[j * bq:(j + 1) * bq, None])
    o_ref[...] = jnp.concatenate(outs, axis=1)


def kernel(Q, K, V, W_Q, b_Q, W_K, b_K, W_V, b_V):
    batch, seq_len, d_model = Q.shape
    d_out = W_Q.shape[1]
    n_heads = d_out // D_K
    q2 = Q.reshape(seq_len, d_model)
    k2 = K.reshape(seq_len, d_model)
    v2 = V.reshape(seq_len, d_model)

    sb = 256
    proj = pl.pallas_call(
        _proj_body,
        grid=(seq_len // sb,),
        in_specs=[
            pl.BlockSpec((sb, d_model), lambda i: (i, 0)),
            pl.BlockSpec((sb, d_model), lambda i: (i, 0)),
            pl.BlockSpec((sb, d_model), lambda i: (i, 0)),
            pl.BlockSpec((d_model, d_out), lambda i: (0, 0)),
            pl.BlockSpec((1, d_out), lambda i: (0, 0)),
            pl.BlockSpec((d_model, d_out), lambda i: (0, 0)),
            pl.BlockSpec((1, d_out), lambda i: (0, 0)),
            pl.BlockSpec((d_model, d_out), lambda i: (0, 0)),
            pl.BlockSpec((1, d_out), lambda i: (0, 0)),
        ],
        out_specs=[
            pl.BlockSpec((sb, d_out), lambda i: (i, 0)),
            pl.BlockSpec((sb, d_out), lambda i: (i, 0)),
            pl.BlockSpec((sb, d_out), lambda i: (i, 0)),
        ],
        out_shape=[jax.ShapeDtypeStruct((seq_len, d_out), jnp.float32)] * 3,
    )
    qs, ks, vs = proj(q2, k2, v2,
                      W_Q, b_Q.reshape(1, d_out),
                      W_K, b_K.reshape(1, d_out),
                      W_V, b_V.reshape(1, d_out))

    bq = 128
    hpp = 2
    nqb = seq_len // bq
    nsplit = 4
    nh2 = n_heads // nsplit
    threshes = []
    for half in range(nsplit):
        off = half * (nh2 // hpp)
        sc_half = pl.pallas_call(
            functools.partial(_scores_body, bq=bq, hpp=hpp),
            grid=(nh2 // hpp, nqb),
            in_specs=[
                pl.BlockSpec((bq, hpp * D_K), lambda h, i, o=off: (i, h + o)),
                pl.BlockSpec((seq_len, hpp * D_K), lambda h, i, o=off: (0, h + o)),
            ],
            out_specs=pl.BlockSpec((hpp, 1, bq, seq_len), lambda h, i: (h, i, 0, 0)),
            out_shape=jax.ShapeDtypeStruct((nh2, nqb, bq, seq_len), jnp.float32),
        )(qs, ks)
        n_rows = nh2 * seq_len
        threshes.append(_sc_threshold(sc_half.reshape(n_rows, seq_len), seq_len, n_rows))
    outs = []
    for half in range(nsplit):
        off = half * (nh2 // hpp)
        t3 = threshes[half].reshape(nh2 * nqb, 1, bq)
        outs.append(pl.pallas_call(
            functools.partial(_finish_body, seq_len=seq_len, bq=bq, hpp=hpp),
            grid=(nh2 // hpp, nqb),
            in_specs=[
                pl.BlockSpec((bq, hpp * D_K), lambda h, i, o=off: (i, h + o)),
                pl.BlockSpec((seq_len, hpp * D_K), lambda h, i, o=off: (0, h + o)),
                pl.BlockSpec((seq_len, hpp * D_V), lambda h, i, o=off: (0, h + o)),
                pl.BlockSpec((1, 1, bq), lambda h, i, n=nqb: (2 * h * n + i, 0, 0)),
                pl.BlockSpec((1, 1, bq), lambda h, i, n=nqb: ((2 * h + 1) * n + i, 0, 0)),
            ],
            out_specs=pl.BlockSpec((bq, hpp * D_V), lambda h, i: (i, h)),
            out_shape=jax.ShapeDtypeStruct((seq_len, nh2 * D_V), jnp.float32),
        )(qs, ks, vs, t3, t3))
    return jnp.concatenate(outs, axis=1).reshape(batch, seq_len, d_out)
